# baseline (speedup 1.0000x reference)
import functools

import jax
import jax.numpy as jnp
from jax import lax
from jax.experimental import pallas as pl
from jax.experimental.pallas import tpu as pltpu
from jax.experimental.pallas import tpu_sc as plsc

NUM_EXPERTS = 64
TOP_K = 8
TOKEN_BLOCK = 512
NC, NS = 2, 16
NW = NC * NS
L = 16
CHUNK_TOKENS = 8192
TOK_PER_W = CHUNK_TOKENS // NW


def _score_body(x_ref, w_ref, s_ref):
    s_ref[...] = jax.nn.sigmoid(
        jnp.dot(x_ref[...], w_ref[...], preferred_element_type=jnp.float32)
    )


def _hist_body(p_ref, h_ref):
    h_ref[...] = jnp.sum(p_ref[...], axis=0, keepdims=True)


def _pack_body(tf_ref, if_ref, top_ref, idx_ref):
    xt = tf_ref[...]
    xi = if_ref[...]
    top_ref[...] = jnp.concatenate([xt[:, 8 * s : 8 * s + 8] for s in range(16)], axis=0)
    idx_ref[...] = jnp.concatenate([xi[:, 8 * s : 8 * s + 8] for s in range(16)], axis=0)


def _vtake(v, idx):
    dn = lax.GatherDimensionNumbers(
        offset_dims=(), collapsed_slice_dims=(0,), start_index_map=(0,)
    )
    return lax.gather(
        v, idx[:, None], dn, (1,), mode=lax.GatherScatterMode.PROMISE_IN_BOUNDS
    )


def _merge16(ak, ai, bk, bi):
    rk = lax.rev(bk, (0,))
    ri = lax.rev(bi, (0,))
    take = (ak > rk) | ((ak == rk) & (ai < ri))
    mk = jnp.where(take, ak, rk)
    mi = jnp.where(take, ai, ri)
    return plsc.sort_key_val(mk, mi, descending=True)


def _route_body(s_hbm, b_hbm, top_hbm, idx_hbm, hist_hbm, slab, biasv, outs, outi, hist):
    wid = lax.axis_index("s") * NC + lax.axis_index("c")
    base = wid * TOK_PER_W
    pltpu.sync_copy(b_hbm, biasv)
    pltpu.sync_copy(s_hbm.at[pl.ds(base, TOK_PER_W)], slab)

    iota = lax.iota(jnp.int32, L)
    lane_lt8 = iota < TOP_K
    ones = jnp.ones((L,), jnp.float32)
    zeros = jnp.zeros((L,), jnp.float32)
    cidx = jnp.maximum(iota - TOP_K, 0)
    for j in range(NUM_EXPERTS // L):
        hist[pl.ds(j * L, L)] = zeros
    biasvs = [biasv[pl.ds(j * L, L)] for j in range(NUM_EXPERTS // L)]
    iotas = [iota + j * L for j in range(NUM_EXPERTS // L)]

    def token(t):
        sk = []
        for j in range(NUM_EXPERTS // L):
            k = slab[t, pl.ds(j * L, L)] + biasvs[j]
            sk.append(plsc.sort_key_val(k, iotas[j], descending=True))
        k01 = _merge16(*sk[0], *sk[1])
        k23 = _merge16(*sk[2], *sk[3])
        bk, bi = _merge16(*k01, *k23)
        tvec = jnp.full((L,), t, jnp.int32)
        g = plsc.load_gather(slab, [tvec, bi])
        g = jnp.where(lane_lt8, g, 0.0)
        gn = g / (jnp.sum(g) + 1e-20)
        plsc.addupdate_scatter(hist, [bi], ones, mask=lane_lt8)
        return gn, bi

    @plsc.parallel_loop(0, TOK_PER_W // 2, 1, unroll=2)
    def pair(p):
        r = p % 16
        jj = p // 16
        t0 = 32 * jj + r
        gn0, bi0 = token(t0)
        gn1, bi1 = token(t0 + 16)
        combs = jnp.where(lane_lt8, gn0, _vtake(gn1, cidx))
        combi = jnp.where(lane_lt8, bi0, _vtake(bi1, cidx))
        off = r * 128 + 16 * jj
        outs[pl.ds(off, L)] = combs
        outi[pl.ds(off, L)] = combi

    pltpu.sync_copy(outs, top_hbm.at[pl.ds(base * TOP_K, TOK_PER_W * TOP_K)])
    pltpu.sync_copy(outi, idx_hbm.at[pl.ds(base * TOP_K, TOK_PER_W * TOP_K)])
    pltpu.sync_copy(hist, hist_hbm.at[wid])


_route = pl.kernel(
    _route_body,
    out_type=[
        jax.ShapeDtypeStruct((CHUNK_TOKENS * TOP_K,), jnp.float32),
        jax.ShapeDtypeStruct((CHUNK_TOKENS * TOP_K,), jnp.int32),
        jax.ShapeDtypeStruct((NW, NUM_EXPERTS), jnp.float32),
    ],
    mesh=plsc.VectorSubcoreMesh(
        core_axis_name="c", subcore_axis_name="s", num_cores=NC, num_subcores=NS
    ),
    compiler_params=pltpu.CompilerParams(needs_layout_passes=False),
    scratch_types=[
        pltpu.VMEM((TOK_PER_W, NUM_EXPERTS), jnp.float32),
        pltpu.VMEM((NUM_EXPERTS,), jnp.float32),
        pltpu.VMEM((TOK_PER_W * TOP_K,), jnp.float32),
        pltpu.VMEM((TOK_PER_W * TOP_K,), jnp.int32),
        pltpu.VMEM((NUM_EXPERTS,), jnp.float32),
    ],
)


@jax.jit
def kernel(x, expert_bias, gate_w):
    tokens, dim = x.shape
    e = gate_w.shape[0]
    w_t = gate_w.T
    t = TOKEN_BLOCK
    tops, idxs, hps = [], [], []
    for h in range(tokens // CHUNK_TOKENS):
        off = h * (CHUNK_TOKENS // t)
        scores_h = pl.pallas_call(
            _score_body,
            grid=(CHUNK_TOKENS // t,),
            in_specs=[
                pl.BlockSpec((t, dim), lambda i, off=off: (i + off, 0)),
                pl.BlockSpec((dim, e), lambda i: (0, 0)),
            ],
            out_specs=pl.BlockSpec((t, e), lambda i: (i, 0)),
            out_shape=jax.ShapeDtypeStruct((CHUNK_TOKENS, e), jnp.float32),
        )(x, w_t)
        top_h, idx_h, hp_h = _route(scores_h, expert_bias)
        tops.append(top_h)
        idxs.append(idx_h)
        hps.append(hp_h)
    hist = pl.pallas_call(
        _hist_body,
        out_shape=jax.ShapeDtypeStruct((1, e), jnp.float32),
    )(jnp.concatenate(hps, axis=0))
    tf = jnp.concatenate([a.reshape(CHUNK_TOKENS * TOP_K // 128, 128) for a in tops])
    itf = jnp.concatenate([a.reshape(CHUNK_TOKENS * TOP_K // 128, 128) for a in idxs])
    pt = 256
    rows = pt * TOP_K // 128
    top, idx = pl.pallas_call(
        _pack_body,
        grid=(tokens // pt,),
        in_specs=[
            pl.BlockSpec((rows, 128), lambda i: (i, 0)),
            pl.BlockSpec((rows, 128), lambda i: (i, 0)),
        ],
        out_specs=[
            pl.BlockSpec((pt, TOP_K), lambda i: (i, 0)),
            pl.BlockSpec((pt, TOP_K), lambda i: (i, 0)),
        ],
        out_shape=[
            jax.ShapeDtypeStruct((tokens, TOP_K), jnp.float32),
            jax.ShapeDtypeStruct((tokens, TOP_K), jnp.int32),
        ],
    )(tf, itf)
    return top, idx, hist.reshape(e)


# speedup vs baseline: 1.2193x; 1.2193x over previous
import functools

import jax
import jax.numpy as jnp
from jax import lax
from jax.experimental import pallas as pl
from jax.experimental.pallas import tpu as pltpu
from jax.experimental.pallas import tpu_sc as plsc

NUM_EXPERTS = 64
TOP_K = 8
TOKEN_BLOCK = 512
NC, NS = 2, 16
NW = NC * NS
L = 16
CHUNK_TOKENS = 8192
TOK_PER_W = CHUNK_TOKENS // NW


def _score_body(x_ref, w_ref, s_ref):
    s_ref[...] = jax.nn.sigmoid(
        jnp.dot(x_ref[...], w_ref[...], preferred_element_type=jnp.float32)
    )


def _hist_body(p_ref, h_ref):
    h_ref[...] = jnp.sum(p_ref[...], axis=0, keepdims=True)


def _vtake(v, idx):
    dn = lax.GatherDimensionNumbers(
        offset_dims=(), collapsed_slice_dims=(0,), start_index_map=(0,)
    )
    return lax.gather(
        v, idx[:, None], dn, (1,), mode=lax.GatherScatterMode.PROMISE_IN_BOUNDS
    )


def _merge16(ak, ai, bk, bi):
    rk = lax.rev(bk, (0,))
    ri = lax.rev(bi, (0,))
    take = (ak > rk) | ((ak == rk) & (ai < ri))
    mk = jnp.where(take, ak, rk)
    mi = jnp.where(take, ai, ri)
    return plsc.sort_key_val(mk, mi, descending=True)


def _route_body(s_hbm, b_hbm, top_hbm, idx_hbm, hist_hbm, slab, biasv, outs, outi, hist):
    wid = lax.axis_index("s") * NC + lax.axis_index("c")
    base = wid * TOK_PER_W
    pltpu.sync_copy(b_hbm, biasv)
    pltpu.sync_copy(s_hbm.at[pl.ds(base, TOK_PER_W)], slab)

    iota = lax.iota(jnp.int32, L)
    lane_lt8 = iota < TOP_K
    ones = jnp.ones((L,), jnp.float32)
    zeros = jnp.zeros((L,), jnp.float32)
    cidx = jnp.maximum(iota - TOP_K, 0)
    for j in range(NUM_EXPERTS // L):
        hist[pl.ds(j * L, L)] = zeros
    biasvs = [biasv[pl.ds(j * L, L)] for j in range(NUM_EXPERTS // L)]
    iotas = [iota + j * L for j in range(NUM_EXPERTS // L)]

    def token(t):
        sk = []
        for j in range(NUM_EXPERTS // L):
            k = slab[t, pl.ds(j * L, L)] + biasvs[j]
            sk.append(plsc.sort_key_val(k, iotas[j], descending=True))
        k01 = _merge16(*sk[0], *sk[1])
        k23 = _merge16(*sk[2], *sk[3])
        bk, bi = _merge16(*k01, *k23)
        tvec = jnp.full((L,), t, jnp.int32)
        g = plsc.load_gather(slab, [tvec, bi])
        g = jnp.where(lane_lt8, g, 0.0)
        gn = g / (jnp.sum(g) + 1e-20)
        plsc.addupdate_scatter(hist, [bi], ones, mask=lane_lt8)
        return gn, bi

    @plsc.parallel_loop(0, TOK_PER_W // 2, 1, unroll=2)
    def pair(p):
        t0 = p * 2
        gn0, bi0 = token(t0)
        gn1, bi1 = token(t0 + 1)
        combs = jnp.where(lane_lt8, gn0, _vtake(gn1, cidx))
        combi = jnp.where(lane_lt8, bi0, _vtake(bi1, cidx))
        outs[pl.ds(p * L, L)] = combs
        outi[pl.ds(p * L, L)] = combi

    pltpu.sync_copy(outs, top_hbm.at[pl.ds(base * TOP_K, TOK_PER_W * TOP_K)])
    pltpu.sync_copy(outi, idx_hbm.at[pl.ds(base * TOP_K, TOK_PER_W * TOP_K)])
    pltpu.sync_copy(hist, hist_hbm.at[wid])


_route = pl.kernel(
    _route_body,
    out_type=[
        jax.ShapeDtypeStruct((CHUNK_TOKENS * TOP_K,), jnp.float32),
        jax.ShapeDtypeStruct((CHUNK_TOKENS * TOP_K,), jnp.int32),
        jax.ShapeDtypeStruct((NW, NUM_EXPERTS), jnp.float32),
    ],
    mesh=plsc.VectorSubcoreMesh(
        core_axis_name="c", subcore_axis_name="s", num_cores=NC, num_subcores=NS
    ),
    compiler_params=pltpu.CompilerParams(needs_layout_passes=False),
    scratch_types=[
        pltpu.VMEM((TOK_PER_W, NUM_EXPERTS), jnp.float32),
        pltpu.VMEM((NUM_EXPERTS,), jnp.float32),
        pltpu.VMEM((TOK_PER_W * TOP_K,), jnp.float32),
        pltpu.VMEM((TOK_PER_W * TOP_K,), jnp.int32),
        pltpu.VMEM((NUM_EXPERTS,), jnp.float32),
    ],
)


@jax.jit
def kernel(x, expert_bias, gate_w):
    tokens, dim = x.shape
    e = gate_w.shape[0]
    w_t = gate_w.T
    t = TOKEN_BLOCK
    tops, idxs, hps = [], [], []
    for h in range(tokens // CHUNK_TOKENS):
        off = h * (CHUNK_TOKENS // t)
        scores_h = pl.pallas_call(
            _score_body,
            grid=(CHUNK_TOKENS // t,),
            in_specs=[
                pl.BlockSpec((t, dim), lambda i, off=off: (i + off, 0)),
                pl.BlockSpec((dim, e), lambda i: (0, 0)),
            ],
            out_specs=pl.BlockSpec((t, e), lambda i: (i, 0)),
            out_shape=jax.ShapeDtypeStruct((CHUNK_TOKENS, e), jnp.float32),
        )(x, w_t)
        top_h, idx_h, hp_h = _route(scores_h, expert_bias)
        tops.append(top_h)
        idxs.append(idx_h)
        hps.append(hp_h)
    hist = pl.pallas_call(
        _hist_body,
        out_shape=jax.ShapeDtypeStruct((1, e), jnp.float32),
    )(jnp.concatenate(hps, axis=0))
    tflat, iflat = lax.optimization_barrier(
        (jnp.concatenate(tops), jnp.concatenate(idxs))
    )
    top = tflat.reshape(tokens, TOP_K)
    idx = iflat.reshape(tokens, TOP_K)
    return top, idx, hist.reshape(e)


# speedup vs baseline: 1.2545x; 1.0289x over previous
import functools

import jax
import jax.numpy as jnp
from jax import lax
from jax.experimental import pallas as pl
from jax.experimental.pallas import tpu as pltpu
from jax.experimental.pallas import tpu_sc as plsc

NUM_EXPERTS = 64
TOP_K = 8
TOKEN_BLOCK = 512
NC, NS = 2, 16
NW = NC * NS
L = 16
CHUNK_TOKENS = 8192
TOK_PER_W = CHUNK_TOKENS // NW


def _score_body(x_ref, w_ref, s_ref):
    s_ref[...] = jax.nn.sigmoid(
        jnp.dot(x_ref[...], w_ref[...], preferred_element_type=jnp.float32)
    )


def _hist_body(p_ref, h_ref):
    h_ref[...] = jnp.sum(p_ref[...], axis=0, keepdims=True)


def _vtake(v, idx):
    dn = lax.GatherDimensionNumbers(
        offset_dims=(), collapsed_slice_dims=(0,), start_index_map=(0,)
    )
    return lax.gather(
        v, idx[:, None], dn, (1,), mode=lax.GatherScatterMode.PROMISE_IN_BOUNDS
    )


def _merge16(ak, ai, bk, bi):
    rk = lax.rev(bk, (0,))
    ri = lax.rev(bi, (0,))
    take = (ak > rk) | ((ak == rk) & (ai < ri))
    mk = jnp.where(take, ak, rk)
    mi = jnp.where(take, ai, ri)
    return plsc.sort_key_val(mk, mi, descending=True)


def _route_body(s_hbm, b_hbm, top_hbm, idx_hbm, hist_hbm, slab, biasv, outs, outi, hist):
    wid = lax.axis_index("s") * NC + lax.axis_index("c")
    base = wid * TOK_PER_W
    pltpu.sync_copy(b_hbm, biasv)
    pltpu.sync_copy(s_hbm.at[pl.ds(base, TOK_PER_W)], slab)

    iota = lax.iota(jnp.int32, L)
    lane_lt8 = iota < TOP_K
    ones = jnp.ones((L,), jnp.float32)
    zeros = jnp.zeros((L,), jnp.float32)
    cidx = jnp.maximum(iota - TOP_K, 0)
    for j in range(NUM_EXPERTS // L):
        hist[pl.ds(j * L, L)] = zeros
    biasvs = [biasv[pl.ds(j * L, L)] for j in range(NUM_EXPERTS // L)]
    iotas = [iota + j * L for j in range(NUM_EXPERTS // L)]

    def token(t):
        sk = []
        for j in range(NUM_EXPERTS // L):
            k = slab[t, pl.ds(j * L, L)] + biasvs[j]
            sk.append(plsc.sort_key_val(k, iotas[j], descending=True))
        k01 = _merge16(*sk[0], *sk[1])
        k23 = _merge16(*sk[2], *sk[3])
        bk, bi = _merge16(*k01, *k23)
        tvec = jnp.full((L,), t, jnp.int32)
        g = plsc.load_gather(slab, [tvec, bi])
        g = jnp.where(lane_lt8, g, 0.0)
        gn = g / (jnp.sum(g) + 1e-20)
        plsc.addupdate_scatter(hist, [bi], ones, mask=lane_lt8)
        return gn, bi

    @plsc.parallel_loop(0, TOK_PER_W // 2, 1, unroll=2)
    def pair(p):
        t0 = p * 2
        gn0, bi0 = token(t0)
        gn1, bi1 = token(t0 + 1)
        combs = jnp.where(lane_lt8, gn0, _vtake(gn1, cidx))
        combi = jnp.where(lane_lt8, bi0, _vtake(bi1, cidx))
        outs[pl.ds(p * L, L)] = combs
        outi[pl.ds(p * L, L)] = combi

    pltpu.sync_copy(outs, top_hbm.at[pl.ds(base * TOP_K, TOK_PER_W * TOP_K)])
    pltpu.sync_copy(outi, idx_hbm.at[pl.ds(base * TOP_K, TOK_PER_W * TOP_K)])
    pltpu.sync_copy(hist, hist_hbm.at[wid])


_route = pl.kernel(
    _route_body,
    out_type=[
        jax.ShapeDtypeStruct((CHUNK_TOKENS * TOP_K,), jnp.float32),
        jax.ShapeDtypeStruct((CHUNK_TOKENS * TOP_K,), jnp.int32),
        jax.ShapeDtypeStruct((NW, NUM_EXPERTS), jnp.float32),
    ],
    mesh=plsc.VectorSubcoreMesh(
        core_axis_name="c", subcore_axis_name="s", num_cores=NC, num_subcores=NS
    ),
    compiler_params=pltpu.CompilerParams(needs_layout_passes=False),
    scratch_types=[
        pltpu.VMEM((TOK_PER_W, NUM_EXPERTS), jnp.float32),
        pltpu.VMEM((NUM_EXPERTS,), jnp.float32),
        pltpu.VMEM((TOK_PER_W * TOP_K,), jnp.float32),
        pltpu.VMEM((TOK_PER_W * TOP_K,), jnp.int32),
        pltpu.VMEM((NUM_EXPERTS,), jnp.float32),
    ],
)


@jax.jit
def kernel(x, expert_bias, gate_w):
    tokens, dim = x.shape
    e = gate_w.shape[0]
    w_t = gate_w.T
    t = TOKEN_BLOCK
    tops, idxs, hps = [], [], []
    for h in range(tokens // CHUNK_TOKENS):
        off = h * (CHUNK_TOKENS // t)
        scores_h = pl.pallas_call(
            _score_body,
            grid=(CHUNK_TOKENS // t,),
            in_specs=[
                pl.BlockSpec((t, dim), lambda i, off=off: (i + off, 0)),
                pl.BlockSpec((dim, e), lambda i: (0, 0)),
            ],
            out_specs=pl.BlockSpec((t, e), lambda i: (i, 0)),
            out_shape=jax.ShapeDtypeStruct((CHUNK_TOKENS, e), jnp.float32),
        )(x, w_t)
        top_h, idx_h, hp_h = _route(scores_h, expert_bias)
        tops.append(top_h)
        idxs.append(idx_h)
        hps.append(hp_h)
    hist = pl.pallas_call(
        _hist_body,
        out_shape=jax.ShapeDtypeStruct((1, e), jnp.float32),
    )(jnp.concatenate(hps, axis=0))
    top = jnp.concatenate(tops).reshape(tokens, TOP_K)
    idx = jnp.concatenate(idxs).reshape(tokens, TOP_K)
    return top, idx, hist.reshape(e)


# speedup vs baseline: 1.2547x; 1.0001x over previous
import functools

import jax
import jax.numpy as jnp
from jax import lax
from jax.experimental import pallas as pl
from jax.experimental.pallas import tpu as pltpu
from jax.experimental.pallas import tpu_sc as plsc

NUM_EXPERTS = 64
TOP_K = 8
TOKEN_BLOCK = 512
NC, NS = 2, 16
NW = NC * NS
L = 16
CHUNK_TOKENS = 8192
TOK_PER_W = CHUNK_TOKENS // NW


def _score_body(x_ref, w_ref, s_ref):
    s_ref[...] = jax.nn.sigmoid(
        jnp.dot(x_ref[...], w_ref[...], preferred_element_type=jnp.float32)
    )


def _hist_body(p_ref, h_ref):
    h_ref[...] = jnp.sum(p_ref[...], axis=0, keepdims=True)


def _vtake(v, idx):
    dn = lax.GatherDimensionNumbers(
        offset_dims=(), collapsed_slice_dims=(0,), start_index_map=(0,)
    )
    return lax.gather(
        v, idx[:, None], dn, (1,), mode=lax.GatherScatterMode.PROMISE_IN_BOUNDS
    )


def _merge16(ak, ai, bk, bi):
    rk = lax.rev(bk, (0,))
    ri = lax.rev(bi, (0,))
    take = (ak > rk) | ((ak == rk) & (ai < ri))
    mk = jnp.where(take, ak, rk)
    mi = jnp.where(take, ai, ri)
    return plsc.sort_key_val(mk, mi, descending=True)


def _route_body(s_hbm, b_hbm, top_hbm, idx_hbm, hist_hbm, slab, biasv, outs, outi, hist):
    wid = lax.axis_index("s") * NC + lax.axis_index("c")
    base = wid * TOK_PER_W
    pltpu.sync_copy(b_hbm, biasv)
    pltpu.sync_copy(s_hbm.at[pl.ds(base, TOK_PER_W)], slab)

    iota = lax.iota(jnp.int32, L)
    lane_lt8 = iota < TOP_K
    ones = jnp.ones((L,), jnp.float32)
    zeros = jnp.zeros((L,), jnp.float32)
    cidx = jnp.maximum(iota - TOP_K, 0)
    for j in range(NUM_EXPERTS // L):
        hist[pl.ds(j * L, L)] = zeros
    biasvs = [biasv[pl.ds(j * L, L)] for j in range(NUM_EXPERTS // L)]
    iotas = [iota + j * L for j in range(NUM_EXPERTS // L)]

    def token(t):
        sk = []
        for j in range(NUM_EXPERTS // L):
            k = slab[t, pl.ds(j * L, L)] + biasvs[j]
            sk.append(plsc.sort_key_val(k, iotas[j], descending=True))
        k01 = _merge16(*sk[0], *sk[1])
        k23 = _merge16(*sk[2], *sk[3])
        bk, bi = _merge16(*k01, *k23)
        tvec = jnp.full((L,), t, jnp.int32)
        g = plsc.load_gather(slab, [tvec, bi])
        g = jnp.where(lane_lt8, g, 0.0)
        gn = g / (jnp.sum(g) + 1e-20)
        plsc.addupdate_scatter(hist, [bi], ones, mask=lane_lt8)
        return gn, bi

    @plsc.parallel_loop(0, TOK_PER_W // 2, 1, unroll=4)
    def pair(p):
        t0 = p * 2
        gn0, bi0 = token(t0)
        gn1, bi1 = token(t0 + 1)
        combs = jnp.where(lane_lt8, gn0, _vtake(gn1, cidx))
        combi = jnp.where(lane_lt8, bi0, _vtake(bi1, cidx))
        outs[pl.ds(p * L, L)] = combs
        outi[pl.ds(p * L, L)] = combi

    pltpu.sync_copy(outs, top_hbm.at[pl.ds(base * TOP_K, TOK_PER_W * TOP_K)])
    pltpu.sync_copy(outi, idx_hbm.at[pl.ds(base * TOP_K, TOK_PER_W * TOP_K)])
    pltpu.sync_copy(hist, hist_hbm.at[wid])


_route = pl.kernel(
    _route_body,
    out_type=[
        jax.ShapeDtypeStruct((CHUNK_TOKENS * TOP_K,), jnp.float32),
        jax.ShapeDtypeStruct((CHUNK_TOKENS * TOP_K,), jnp.int32),
        jax.ShapeDtypeStruct((NW, NUM_EXPERTS), jnp.float32),
    ],
    mesh=plsc.VectorSubcoreMesh(
        core_axis_name="c", subcore_axis_name="s", num_cores=NC, num_subcores=NS
    ),
    compiler_params=pltpu.CompilerParams(needs_layout_passes=False),
    scratch_types=[
        pltpu.VMEM((TOK_PER_W, NUM_EXPERTS), jnp.float32),
        pltpu.VMEM((NUM_EXPERTS,), jnp.float32),
        pltpu.VMEM((TOK_PER_W * TOP_K,), jnp.float32),
        pltpu.VMEM((TOK_PER_W * TOP_K,), jnp.int32),
        pltpu.VMEM((NUM_EXPERTS,), jnp.float32),
    ],
)


@jax.jit
def kernel(x, expert_bias, gate_w):
    tokens, dim = x.shape
    e = gate_w.shape[0]
    w_t = gate_w.T
    t = TOKEN_BLOCK
    tops, idxs, hps = [], [], []
    for h in range(tokens // CHUNK_TOKENS):
        off = h * (CHUNK_TOKENS // t)
        scores_h = pl.pallas_call(
            _score_body,
            grid=(CHUNK_TOKENS // t,),
            in_specs=[
                pl.BlockSpec((t, dim), lambda i, off=off: (i + off, 0)),
                pl.BlockSpec((dim, e), lambda i: (0, 0)),
            ],
            out_specs=pl.BlockSpec((t, e), lambda i: (i, 0)),
            out_shape=jax.ShapeDtypeStruct((CHUNK_TOKENS, e), jnp.float32),
        )(x, w_t)
        top_h, idx_h, hp_h = _route(scores_h, expert_bias)
        tops.append(top_h)
        idxs.append(idx_h)
        hps.append(hp_h)
    hist = pl.pallas_call(
        _hist_body,
        out_shape=jax.ShapeDtypeStruct((1, e), jnp.float32),
    )(jnp.concatenate(hps, axis=0))
    top = jnp.concatenate(tops).reshape(tokens, TOP_K)
    idx = jnp.concatenate(idxs).reshape(tokens, TOP_K)
    return top, idx, hist.reshape(e)
